# Initial kernel scaffold; baseline (speedup 1.0000x reference)
import jax
import jax.numpy as jnp
from jax import lax
from jax.experimental import pallas as pl
from jax.experimental.pallas import tpu as pltpu
from jax.experimental.pallas import tpu_sc as plsc

N = 10000
D = 128
E = 320000
G = 128
NP = 10240
NT = 16
CHUNK = 128
NCHUNK = 157
EP = NT * NCHUNK * CHUNK
ROWS_PER_TILE = NP // NT
DEG_ROWS = NP // 128

_f32 = jnp.float32
_i32 = jnp.int32


def _mesh():
    return plsc.VectorSubcoreMesh(core_axis_name="c", subcore_axis_name="s")


def _zero_rows(ref, nrows):
    zero16 = jnp.zeros((16,), _f32)

    def body(i, carry):
        for k in range(8):
            ref[i, pl.ds(k * 16, 16)] = zero16
        return carry

    lax.fori_loop(0, nrows, body, 0)


def _deg_body(dsts_hbm, rowid_hbm, deg_out, dst_v, deg2_v, rowid_v, deg_sh):
    c = lax.axis_index("c")
    s = lax.axis_index("s")
    _zero_rows(deg2_v, DEG_ROWS)
    pltpu.sync_copy(deg2_v.at[pl.ds(s * 5, 5)], deg_sh.at[pl.ds(s * 5, 5)])
    plsc.subcore_barrier()

    pltpu.sync_copy(dsts_hbm.at[c, s], dst_v)
    pltpu.sync_copy(rowid_hbm, rowid_v)
    ones16 = jnp.ones((16,), _f32)

    def body(j, carry):
        for k in range(8):
            idx = dst_v[j, pl.ds(k * 16, 16)]
            r = lax.shift_right_logical(idx, 7)
            col = lax.bitwise_and(idx, 127)
            plsc.addupdate_scatter(deg2_v, [r, col], ones16)
        return carry

    lax.fori_loop(0, NCHUNK, body, 0)
    pltpu.sync_copy(deg2_v, deg_sh.at[rowid_v], add=True)
    plsc.subcore_barrier()
    pltpu.sync_copy(deg_sh.at[pl.ds(s * 5, 5)], deg_out.at[c, pl.ds(s * 5, 5)])


def _deg_call(dsts, rowid):
    return pl.kernel(
        _deg_body,
        out_type=jax.ShapeDtypeStruct((2, DEG_ROWS, 128), _f32),
        mesh=_mesh(),
        scratch_types=[
            pltpu.VMEM((NCHUNK, CHUNK), _i32),
            pltpu.VMEM((DEG_ROWS, 128), _f32),
            pltpu.VMEM((DEG_ROWS,), _i32),
            pltpu.VMEM_SHARED((DEG_ROWS, 128), _f32),
        ],
    )(dsts, rowid)


def _scatter_body(g_hbm, srcs_hbm, dsts_hbm, acc_out, src_v, dst_v, rowbuf,
                  acc_sh, sem):
    c = lax.axis_index("c")
    s = lax.axis_index("s")
    _zero_rows(rowbuf, CHUNK)
    for r in range(ROWS_PER_TILE // CHUNK):
        pltpu.sync_copy(rowbuf,
                        acc_sh.at[pl.ds(s * ROWS_PER_TILE + r * CHUNK, CHUNK)])
    plsc.subcore_barrier()

    pltpu.sync_copy(srcs_hbm.at[c, s], src_v)
    pltpu.sync_copy(dsts_hbm.at[c, s], dst_v)

    def body(j, carry):
        pltpu.async_copy(g_hbm.at[src_v.at[j]], rowbuf, sem).wait()
        pltpu.sync_copy(rowbuf, acc_sh.at[dst_v.at[j]], add=True)
        return carry

    lax.fori_loop(0, NCHUNK, body, 0)
    plsc.subcore_barrier()
    pltpu.sync_copy(acc_sh.at[pl.ds(s * ROWS_PER_TILE, ROWS_PER_TILE)],
                    acc_out.at[c, pl.ds(s * ROWS_PER_TILE, ROWS_PER_TILE)])


def _scatter_call(g_all, srcs, dsts):
    return pl.kernel(
        _scatter_body,
        out_type=jax.ShapeDtypeStruct((2, NP, D), _f32),
        mesh=_mesh(),
        scratch_types=[
            pltpu.VMEM((NCHUNK, CHUNK), _i32),
            pltpu.VMEM((NCHUNK, CHUNK), _i32),
            pltpu.VMEM((CHUNK, D), _f32),
            pltpu.VMEM_SHARED((NP, D), _f32),
            pltpu.SemaphoreType.DMA,
        ],
    )(g_all, srcs, dsts)


def _dis_body(deg_ref, dis_ref):
    deg = deg_ref[0]
    node = (lax.broadcasted_iota(_i32, (DEG_ROWS, 128), 0) * 128
            + lax.broadcasted_iota(_i32, (DEG_ROWS, 128), 1))
    dis_ref[0] = jnp.where(node < N, lax.rsqrt(deg + 1.0), 0.0)


def _dis_call(deg2d):
    return pl.pallas_call(
        _dis_body,
        grid=(2,),
        in_specs=[pl.BlockSpec((1, DEG_ROWS, 128), lambda c: (c, 0, 0))],
        out_specs=pl.BlockSpec((1, DEG_ROWS, 128), lambda c: (c, 0, 0)),
        out_shape=jax.ShapeDtypeStruct((2, DEG_ROWS, 128), _f32),
    )(deg2d)


_BLK = 512


def _g_body(x_ref, dis_ref, w_ref, g_ref):
    g_ref[0] = jnp.dot(x_ref[0] * dis_ref[0], w_ref[0],
                       preferred_element_type=_f32)


def _g_call(x_all, dis_col, w_all):
    return pl.pallas_call(
        _g_body,
        grid=(2, NP // _BLK),
        in_specs=[
            pl.BlockSpec((1, _BLK, D), lambda c, j: (c, j, 0)),
            pl.BlockSpec((1, _BLK, 1), lambda c, j: (c, j, 0)),
            pl.BlockSpec((1, D, D), lambda c, j: (c, 0, 0)),
        ],
        out_specs=pl.BlockSpec((1, _BLK, D), lambda c, j: (c, j, 0)),
        out_shape=jax.ShapeDtypeStruct((2, NP, D), _f32),
    )(x_all, dis_col, w_all)


def _pool_body(acc_ref, g_ref, dis_ref, b_ref, batch_ref, ss_ref, cnt_ref):
    j = pl.program_id(1)
    y = dis_ref[0] * (acc_ref[0] + g_ref[0]) + b_ref[0]
    y = jnp.where(y >= 0, y, 0.01 * y)
    gid = lax.broadcasted_iota(_i32, (1, G), 1)
    oh = (batch_ref[0] == gid).astype(_f32)
    ss = lax.dot_general(oh, y, (((0,), (0,)), ((), ())),
                         preferred_element_type=_f32)
    cn = lax.dot_general(oh, jnp.ones((_BLK, 1), _f32),
                         (((0,), (0,)), ((), ())),
                         preferred_element_type=_f32)

    @pl.when(j == 0)
    def _():
        ss_ref[0] = jnp.zeros((G, D), _f32)
        cnt_ref[0] = jnp.zeros((G, 1), _f32)

    ss_ref[0] += ss
    cnt_ref[0] += cn


def _pool_call(acc, g, dis_col, b_all, batch_all):
    return pl.pallas_call(
        _pool_body,
        grid=(2, NP // _BLK),
        in_specs=[
            pl.BlockSpec((1, _BLK, D), lambda c, j: (c, j, 0)),
            pl.BlockSpec((1, _BLK, D), lambda c, j: (c, j, 0)),
            pl.BlockSpec((1, _BLK, 1), lambda c, j: (c, j, 0)),
            pl.BlockSpec((1, 1, D), lambda c, j: (c, 0, 0)),
            pl.BlockSpec((1, _BLK, 1), lambda c, j: (c, j, 0)),
        ],
        out_specs=[
            pl.BlockSpec((1, G, D), lambda c, j: (c, 0, 0)),
            pl.BlockSpec((1, G, 1), lambda c, j: (c, 0, 0)),
        ],
        out_shape=[
            jax.ShapeDtypeStruct((2, G, D), _f32),
            jax.ShapeDtypeStruct((2, G, 1), _f32),
        ],
    )(acc, g, dis_col, b_all, batch_all)


def _head_body(ss_ref, cnt_ref, fcw_ref, fcb_ref, fw_ref, fb_ref, out_ref):
    o = None
    for cidx in range(2):
        m = ss_ref[cidx] / jnp.maximum(cnt_ref[cidx], 1.0)
        z = jnp.dot(m, fcw_ref[cidx], preferred_element_type=_f32) \
            + fcb_ref[cidx]
        z = jnp.where(z >= 0, z, 0.01 * z)
        fw = fw_ref[pl.ds(cidx * D, D), :]
        contrib = jnp.dot(z, fw, preferred_element_type=_f32)
        o = contrib if o is None else o + contrib
    out_ref[...] = o + fb_ref[...]


def _head_call(ss, cnt, fcw_all, fcb_all, final_w, final_b):
    return pl.pallas_call(
        _head_body,
        out_shape=jax.ShapeDtypeStruct((G, 1), _f32),
    )(ss, cnt, fcw_all, fcb_all, final_w, final_b)


def kernel(pro1_x, pro1_edge_index, pro1_batch, pro2_x, pro2_edge_index,
           pro2_batch, W1, b1, fc1_w, fc1_b, W2, b2, fc2_w, fc2_b,
           final_w, final_b):
    pad = jnp.full((EP - E,), N, _i32)
    s1 = jnp.concatenate([pro1_edge_index[0], pad])
    s2 = jnp.concatenate([pro2_edge_index[0], pad]) + NP
    d1 = jnp.concatenate([pro1_edge_index[1], pad])
    d2 = jnp.concatenate([pro2_edge_index[1], pad])
    srcs = jnp.stack([s1, s2]).reshape(2, NT, NCHUNK, CHUNK)
    dsts = jnp.stack([d1, d2]).reshape(2, NT, NCHUNK, CHUNK)
    rowid = jnp.arange(DEG_ROWS, dtype=_i32)

    x_all = jnp.pad(jnp.stack([pro1_x, pro2_x]), ((0, 0), (0, NP - N), (0, 0)))
    w_all = jnp.stack([W1, W2])
    b_all = jnp.stack([b1, b2]).reshape(2, 1, D)
    batch_all = jnp.pad(jnp.stack([pro1_batch, pro2_batch]),
                        ((0, 0), (0, NP - N)),
                        constant_values=G).reshape(2, NP, 1)
    fcw_all = jnp.stack([fc1_w, fc2_w])
    fcb_all = jnp.stack([fc1_b, fc2_b]).reshape(2, 1, D)
    fb = final_b.reshape(1, 1)

    deg2d = _deg_call(dsts, rowid)
    dis_col = _dis_call(deg2d).reshape(2, NP, 1)
    g = _g_call(x_all, dis_col, w_all)
    g_all = g.reshape(2 * NP, D)
    acc = _scatter_call(g_all, srcs, dsts)
    ss, cnt = _pool_call(acc, g, dis_col, b_all, batch_all)
    return _head_call(ss, cnt, fcw_all, fcb_all, final_w, fb)


# speedup vs baseline: 13.6133x; 13.6133x over previous
import jax
import jax.numpy as jnp
from jax import lax
from jax.experimental import pallas as pl
from jax.experimental.pallas import tpu as pltpu
from jax.experimental.pallas import tpu_sc as plsc

N = 10000
D = 128
E = 320000
G = 128
NP = 10240
NT = 16
CHUNK = 128
NCHUNK = 160
SLAB = 80
EP = NT * NCHUNK * CHUNK
ROWS_PER_TILE = NP // NT
DEG_ROWS = NP // 128

_f32 = jnp.float32
_i32 = jnp.int32


def _mesh():
    return plsc.VectorSubcoreMesh(core_axis_name="c", subcore_axis_name="s")


def _zero_rows(ref, nrows):
    zero16 = jnp.zeros((16,), _f32)

    def body(i, carry):
        for k in range(8):
            ref[i, pl.ds(k * 16, 16)] = zero16
        return carry

    lax.fori_loop(0, nrows, body, 0)


def _deg_body(dsts_hbm, rowid_hbm, deg_out, dst_v, deg1_v, deg2_v, rowid_v,
              deg_sh):
    c = lax.axis_index("c")
    s = lax.axis_index("s")
    zero16 = jnp.zeros((16,), _f32)

    def zbody(i, carry):
        deg1_v[pl.ds(i * 16, 16)] = zero16
        return carry

    lax.fori_loop(0, NP // 16, zbody, 0)
    _zero_rows(deg2_v, DEG_ROWS)

    @pl.when(s < 10)
    def _():
        pltpu.sync_copy(deg2_v.at[pl.ds(s * 8, 8)], deg_sh.at[pl.ds(s * 8, 8)])

    plsc.subcore_barrier()

    pltpu.sync_copy(dsts_hbm.at[c, s], dst_v)
    pltpu.sync_copy(rowid_hbm, rowid_v)
    ones16 = jnp.ones((16,), _f32)

    def body(j, carry):
        for k in range(8):
            idx = dst_v[j, pl.ds(k * 16, 16)]
            plsc.addupdate_scatter(deg1_v, [idx], ones16)
        return carry

    lax.fori_loop(0, NCHUNK, body, 0)

    def rbody(r, carry):
        for k in range(8):
            deg2_v[r, pl.ds(k * 16, 16)] = deg1_v[pl.ds(r * 128 + k * 16, 16)]
        return carry

    lax.fori_loop(0, DEG_ROWS, rbody, 0)
    pltpu.sync_copy(deg2_v, deg_sh.at[rowid_v], add=True)
    plsc.subcore_barrier()

    @pl.when(s < 10)
    def _():
        pltpu.sync_copy(deg_sh.at[pl.ds(s * 8, 8)],
                        deg_out.at[c, pl.ds(s * 8, 8)])


def _deg_call(dsts, rowid):
    return pl.kernel(
        _deg_body,
        out_type=jax.ShapeDtypeStruct((2, DEG_ROWS, 128), _f32),
        mesh=_mesh(),
        scratch_types=[
            pltpu.VMEM((NCHUNK, CHUNK), _i32),
            pltpu.VMEM((NP,), _f32),
            pltpu.VMEM((DEG_ROWS, 128), _f32),
            pltpu.VMEM((DEG_ROWS,), _i32),
            pltpu.VMEM_SHARED((DEG_ROWS, 128), _f32),
        ],
        compiler_params=pltpu.CompilerParams(needs_layout_passes=False),
    )(dsts, rowid)


def _scatter_body(g_hbm, srcs_hbm, dsts_hbm, acc_out, src_v, dst_v, rowbuf,
                  acc_sh, sem):
    c = lax.axis_index("c")
    s = lax.axis_index("s")
    _zero_rows(rowbuf, CHUNK)
    for r in range(ROWS_PER_TILE // CHUNK):
        pltpu.sync_copy(rowbuf,
                        acc_sh.at[pl.ds(s * ROWS_PER_TILE + r * CHUNK, CHUNK)])
    plsc.subcore_barrier()

    def body(j, carry):
        pltpu.async_copy(g_hbm.at[src_v.at[j]], rowbuf, sem).wait()
        pltpu.sync_copy(rowbuf, acc_sh.at[dst_v.at[j]], add=True)
        return carry

    for p in range(NCHUNK // SLAB):
        pltpu.sync_copy(srcs_hbm.at[c, s, pl.ds(p * SLAB, SLAB)], src_v)
        pltpu.sync_copy(dsts_hbm.at[c, s, pl.ds(p * SLAB, SLAB)], dst_v)
        lax.fori_loop(0, SLAB, body, 0)
    plsc.subcore_barrier()
    pltpu.sync_copy(acc_sh.at[pl.ds(s * ROWS_PER_TILE, ROWS_PER_TILE)],
                    acc_out.at[c, pl.ds(s * ROWS_PER_TILE, ROWS_PER_TILE)])


def _scatter_call(g_all, srcs, dsts):
    return pl.kernel(
        _scatter_body,
        out_type=jax.ShapeDtypeStruct((2, NP, D), _f32),
        mesh=_mesh(),
        scratch_types=[
            pltpu.VMEM((SLAB, CHUNK), _i32),
            pltpu.VMEM((SLAB, CHUNK), _i32),
            pltpu.VMEM((CHUNK, D), _f32),
            pltpu.VMEM_SHARED((NP, D), _f32),
            pltpu.SemaphoreType.DMA,
        ],
        compiler_params=pltpu.CompilerParams(needs_layout_passes=False),
    )(g_all, srcs, dsts)


def _dis_body(deg_ref, dis_ref):
    deg = deg_ref[0]
    node = (lax.broadcasted_iota(_i32, (DEG_ROWS, 128), 0) * 128
            + lax.broadcasted_iota(_i32, (DEG_ROWS, 128), 1))
    dis_ref[0] = jnp.where(node < N, lax.rsqrt(deg + 1.0), 0.0)


def _dis_call(deg2d):
    return pl.pallas_call(
        _dis_body,
        grid=(2,),
        in_specs=[pl.BlockSpec((1, DEG_ROWS, 128), lambda c: (c, 0, 0))],
        out_specs=pl.BlockSpec((1, DEG_ROWS, 128), lambda c: (c, 0, 0)),
        out_shape=jax.ShapeDtypeStruct((2, DEG_ROWS, 128), _f32),
    )(deg2d)


_BLK = 512


def _g_body(x_ref, dis_ref, w_ref, g_ref):
    g_ref[0] = jnp.dot(x_ref[0] * dis_ref[0], w_ref[0],
                       preferred_element_type=_f32)


def _g_call(x_all, dis_col, w_all):
    return pl.pallas_call(
        _g_body,
        grid=(2, NP // _BLK),
        in_specs=[
            pl.BlockSpec((1, _BLK, D), lambda c, j: (c, j, 0)),
            pl.BlockSpec((1, _BLK, 1), lambda c, j: (c, j, 0)),
            pl.BlockSpec((1, D, D), lambda c, j: (c, 0, 0)),
        ],
        out_specs=pl.BlockSpec((1, _BLK, D), lambda c, j: (c, j, 0)),
        out_shape=jax.ShapeDtypeStruct((2, NP, D), _f32),
    )(x_all, dis_col, w_all)


def _pool_body(acc_ref, g_ref, dis_ref, b_ref, batch_ref, ss_ref, cnt_ref):
    j = pl.program_id(1)
    y = dis_ref[0] * (acc_ref[0] + g_ref[0]) + b_ref[0]
    y = jnp.where(y >= 0, y, 0.01 * y)
    gid = lax.broadcasted_iota(_i32, (1, G), 1)
    oh = (batch_ref[0] == gid).astype(_f32)
    ss = lax.dot_general(oh, y, (((0,), (0,)), ((), ())),
                         preferred_element_type=_f32)
    cn = lax.dot_general(oh, jnp.ones((_BLK, 1), _f32),
                         (((0,), (0,)), ((), ())),
                         preferred_element_type=_f32)

    @pl.when(j == 0)
    def _():
        ss_ref[0] = jnp.zeros((G, D), _f32)
        cnt_ref[0] = jnp.zeros((G, 1), _f32)

    ss_ref[0] += ss
    cnt_ref[0] += cn


def _pool_call(acc, g, dis_col, b_all, batch_all):
    return pl.pallas_call(
        _pool_body,
        grid=(2, NP // _BLK),
        in_specs=[
            pl.BlockSpec((1, _BLK, D), lambda c, j: (c, j, 0)),
            pl.BlockSpec((1, _BLK, D), lambda c, j: (c, j, 0)),
            pl.BlockSpec((1, _BLK, 1), lambda c, j: (c, j, 0)),
            pl.BlockSpec((1, 1, D), lambda c, j: (c, 0, 0)),
            pl.BlockSpec((1, _BLK, 1), lambda c, j: (c, j, 0)),
        ],
        out_specs=[
            pl.BlockSpec((1, G, D), lambda c, j: (c, 0, 0)),
            pl.BlockSpec((1, G, 1), lambda c, j: (c, 0, 0)),
        ],
        out_shape=[
            jax.ShapeDtypeStruct((2, G, D), _f32),
            jax.ShapeDtypeStruct((2, G, 1), _f32),
        ],
    )(acc, g, dis_col, b_all, batch_all)


def _head_body(ss_ref, cnt_ref, fcw_ref, fcb_ref, fw_ref, fb_ref, out_ref):
    o = None
    for cidx in range(2):
        m = ss_ref[cidx] / jnp.maximum(cnt_ref[cidx], 1.0)
        z = jnp.dot(m, fcw_ref[cidx], preferred_element_type=_f32) \
            + fcb_ref[cidx]
        z = jnp.where(z >= 0, z, 0.01 * z)
        fw = fw_ref[pl.ds(cidx * D, D), :]
        contrib = jnp.dot(z, fw, preferred_element_type=_f32)
        o = contrib if o is None else o + contrib
    out_ref[...] = o + fb_ref[...]


def _head_call(ss, cnt, fcw_all, fcb_all, final_w, final_b):
    return pl.pallas_call(
        _head_body,
        out_shape=jax.ShapeDtypeStruct((G, 1), _f32),
    )(ss, cnt, fcw_all, fcb_all, final_w, final_b)


def kernel(pro1_x, pro1_edge_index, pro1_batch, pro2_x, pro2_edge_index,
           pro2_batch, W1, b1, fc1_w, fc1_b, W2, b2, fc2_w, fc2_b,
           final_w, final_b):
    pad = jnp.full((EP - E,), N, _i32)
    s1 = jnp.concatenate([pro1_edge_index[0], pad])
    s2 = jnp.concatenate([pro2_edge_index[0], pad]) + NP
    d1 = jnp.concatenate([pro1_edge_index[1], pad])
    d2 = jnp.concatenate([pro2_edge_index[1], pad])
    srcs = jnp.stack([s1, s2]).reshape(2, NT, NCHUNK, CHUNK)
    dsts = jnp.stack([d1, d2]).reshape(2, NT, NCHUNK, CHUNK)
    rowid = jnp.arange(DEG_ROWS, dtype=_i32)

    x_all = jnp.pad(jnp.stack([pro1_x, pro2_x]), ((0, 0), (0, NP - N), (0, 0)))
    w_all = jnp.stack([W1, W2])
    b_all = jnp.stack([b1, b2]).reshape(2, 1, D)
    batch_all = jnp.pad(jnp.stack([pro1_batch, pro2_batch]),
                        ((0, 0), (0, NP - N)),
                        constant_values=G).reshape(2, NP, 1)
    fcw_all = jnp.stack([fc1_w, fc2_w])
    fcb_all = jnp.stack([fc1_b, fc2_b]).reshape(2, 1, D)
    fb = final_b.reshape(1, 1)

    deg2d = _deg_call(dsts, rowid)
    dis_col = _dis_call(deg2d).reshape(2, NP, 1)
    g = _g_call(x_all, dis_col, w_all)
    g_all = g.reshape(2 * NP, D)
    acc = _scatter_call(g_all, srcs, dsts)
    ss, cnt = _pool_call(acc, g, dis_col, b_all, batch_all)
    return _head_call(ss, cnt, fcw_all, fcb_all, final_w, fb)


# speedup vs baseline: 15.8791x; 1.1664x over previous
import jax
import jax.numpy as jnp
from jax import lax
from jax.experimental import pallas as pl
from jax.experimental.pallas import tpu as pltpu
from jax.experimental.pallas import tpu_sc as plsc

N = 10000
D = 128
E = 320000
G = 128
NP = 10240
NT = 16
CHUNK = 128
NCHUNK = 160
SLAB = 40
EP = NT * NCHUNK * CHUNK
ROWS_PER_TILE = NP // NT
DEG_ROWS = NP // 128

_f32 = jnp.float32
_i32 = jnp.int32


def _mesh():
    return plsc.VectorSubcoreMesh(core_axis_name="c", subcore_axis_name="s")


def _zero_rows(ref, nrows):
    zero16 = jnp.zeros((16,), _f32)

    def body(i, carry):
        for k in range(8):
            ref[i, pl.ds(k * 16, 16)] = zero16
        return carry

    lax.fori_loop(0, nrows, body, 0)


def _deg_body(dsts_hbm, rowid_hbm, deg_out, dst_v, deg1_v, deg2_v, rowid_v,
              deg_sh):
    c = lax.axis_index("c")
    s = lax.axis_index("s")
    zero16 = jnp.zeros((16,), _f32)

    def zbody(i, carry):
        deg1_v[pl.ds(i * 16, 16)] = zero16
        return carry

    lax.fori_loop(0, NP // 16, zbody, 0)
    _zero_rows(deg2_v, DEG_ROWS)

    @pl.when(s < 10)
    def _():
        pltpu.sync_copy(deg2_v.at[pl.ds(s * 8, 8)], deg_sh.at[pl.ds(s * 8, 8)])

    plsc.subcore_barrier()

    pltpu.sync_copy(dsts_hbm.at[c, s], dst_v)
    pltpu.sync_copy(rowid_hbm, rowid_v)
    ones16 = jnp.ones((16,), _f32)

    def body(j, carry):
        for k in range(8):
            idx = dst_v[j, pl.ds(k * 16, 16)]
            plsc.addupdate_scatter(deg1_v, [idx], ones16)
        return carry

    lax.fori_loop(0, NCHUNK, body, 0)

    def rbody(r, carry):
        for k in range(8):
            deg2_v[r, pl.ds(k * 16, 16)] = deg1_v[pl.ds(r * 128 + k * 16, 16)]
        return carry

    lax.fori_loop(0, DEG_ROWS, rbody, 0)
    pltpu.sync_copy(deg2_v, deg_sh.at[rowid_v], add=True)
    plsc.subcore_barrier()

    @pl.when(s < 10)
    def _():
        pltpu.sync_copy(deg_sh.at[pl.ds(s * 8, 8)],
                        deg_out.at[c, pl.ds(s * 8, 8)])


def _deg_call(dsts, rowid):
    return pl.kernel(
        _deg_body,
        out_type=jax.ShapeDtypeStruct((2, DEG_ROWS, 128), _f32),
        mesh=_mesh(),
        scratch_types=[
            pltpu.VMEM((NCHUNK, CHUNK), _i32),
            pltpu.VMEM((NP,), _f32),
            pltpu.VMEM((DEG_ROWS, 128), _f32),
            pltpu.VMEM((DEG_ROWS,), _i32),
            pltpu.VMEM_SHARED((DEG_ROWS, 128), _f32),
        ],
        compiler_params=pltpu.CompilerParams(needs_layout_passes=False),
    )(dsts, rowid)


def _scatter_body(g_hbm, srcs_hbm, dsts_hbm, acc_out, src_v, dst_v, row_a,
                  row_b, acc_sh, sem_a, sem_b):
    c = lax.axis_index("c")
    s = lax.axis_index("s")
    _zero_rows(row_a, CHUNK)
    for r in range(ROWS_PER_TILE // CHUNK):
        pltpu.sync_copy(row_a,
                        acc_sh.at[pl.ds(s * ROWS_PER_TILE + r * CHUNK, CHUNK)])
    plsc.subcore_barrier()

    def pair(i, carry):
        j = 2 * i
        pltpu.async_copy(g_hbm.at[src_v.at[j + 1]], row_b, sem_b)
        pltpu.make_async_copy(g_hbm.at[src_v.at[j]], row_a, sem_a).wait()
        pltpu.sync_copy(row_a, acc_sh.at[dst_v.at[j]], add=True)

        @pl.when(i < SLAB // 2 - 1)
        def _():
            pltpu.async_copy(g_hbm.at[src_v.at[j + 2]], row_a, sem_a)

        pltpu.make_async_copy(g_hbm.at[src_v.at[j + 1]], row_b, sem_b).wait()
        pltpu.sync_copy(row_b, acc_sh.at[dst_v.at[j + 1]], add=True)
        return carry

    for p in range(NCHUNK // SLAB):
        pltpu.sync_copy(srcs_hbm.at[c, s, pl.ds(p * SLAB, SLAB)], src_v)
        pltpu.sync_copy(dsts_hbm.at[c, s, pl.ds(p * SLAB, SLAB)], dst_v)
        pltpu.async_copy(g_hbm.at[src_v.at[0]], row_a, sem_a)
        lax.fori_loop(0, SLAB // 2, pair, 0)
    plsc.subcore_barrier()
    pltpu.sync_copy(acc_sh.at[pl.ds(s * ROWS_PER_TILE, ROWS_PER_TILE)],
                    acc_out.at[c, pl.ds(s * ROWS_PER_TILE, ROWS_PER_TILE)])


def _scatter_call(g_all, srcs, dsts):
    return pl.kernel(
        _scatter_body,
        out_type=jax.ShapeDtypeStruct((2, NP, D), _f32),
        mesh=_mesh(),
        scratch_types=[
            pltpu.VMEM((SLAB, CHUNK), _i32),
            pltpu.VMEM((SLAB, CHUNK), _i32),
            pltpu.VMEM((CHUNK, D), _f32),
            pltpu.VMEM((CHUNK, D), _f32),
            pltpu.VMEM_SHARED((NP, D), _f32),
            pltpu.SemaphoreType.DMA,
            pltpu.SemaphoreType.DMA,
        ],
        compiler_params=pltpu.CompilerParams(needs_layout_passes=False),
    )(g_all, srcs, dsts)


def _dis_body(deg_ref, dis_ref):
    deg = deg_ref[0]
    node = (lax.broadcasted_iota(_i32, (DEG_ROWS, 128), 0) * 128
            + lax.broadcasted_iota(_i32, (DEG_ROWS, 128), 1))
    dis_ref[0] = jnp.where(node < N, lax.rsqrt(deg + 1.0), 0.0)


def _dis_call(deg2d):
    return pl.pallas_call(
        _dis_body,
        grid=(2,),
        in_specs=[pl.BlockSpec((1, DEG_ROWS, 128), lambda c: (c, 0, 0))],
        out_specs=pl.BlockSpec((1, DEG_ROWS, 128), lambda c: (c, 0, 0)),
        out_shape=jax.ShapeDtypeStruct((2, DEG_ROWS, 128), _f32),
    )(deg2d)


_BLK = 512


def _g_body(x_ref, dis_ref, w_ref, g_ref):
    g_ref[0] = jnp.dot(x_ref[0] * dis_ref[0], w_ref[0],
                       preferred_element_type=_f32)


def _g_call(x_all, dis_col, w_all):
    return pl.pallas_call(
        _g_body,
        grid=(2, NP // _BLK),
        in_specs=[
            pl.BlockSpec((1, _BLK, D), lambda c, j: (c, j, 0)),
            pl.BlockSpec((1, _BLK, 1), lambda c, j: (c, j, 0)),
            pl.BlockSpec((1, D, D), lambda c, j: (c, 0, 0)),
        ],
        out_specs=pl.BlockSpec((1, _BLK, D), lambda c, j: (c, j, 0)),
        out_shape=jax.ShapeDtypeStruct((2, NP, D), _f32),
    )(x_all, dis_col, w_all)


def _pool_body(acc_ref, g_ref, dis_ref, b_ref, batch_ref, ss_ref, cnt_ref):
    j = pl.program_id(1)
    y = dis_ref[0] * (acc_ref[0] + g_ref[0]) + b_ref[0]
    y = jnp.where(y >= 0, y, 0.01 * y)
    gid = lax.broadcasted_iota(_i32, (1, G), 1)
    oh = (batch_ref[0] == gid).astype(_f32)
    ss = lax.dot_general(oh, y, (((0,), (0,)), ((), ())),
                         preferred_element_type=_f32)
    cn = lax.dot_general(oh, jnp.ones((_BLK, 1), _f32),
                         (((0,), (0,)), ((), ())),
                         preferred_element_type=_f32)

    @pl.when(j == 0)
    def _():
        ss_ref[0] = jnp.zeros((G, D), _f32)
        cnt_ref[0] = jnp.zeros((G, 1), _f32)

    ss_ref[0] += ss
    cnt_ref[0] += cn


def _pool_call(acc, g, dis_col, b_all, batch_all):
    return pl.pallas_call(
        _pool_body,
        grid=(2, NP // _BLK),
        in_specs=[
            pl.BlockSpec((1, _BLK, D), lambda c, j: (c, j, 0)),
            pl.BlockSpec((1, _BLK, D), lambda c, j: (c, j, 0)),
            pl.BlockSpec((1, _BLK, 1), lambda c, j: (c, j, 0)),
            pl.BlockSpec((1, 1, D), lambda c, j: (c, 0, 0)),
            pl.BlockSpec((1, _BLK, 1), lambda c, j: (c, j, 0)),
        ],
        out_specs=[
            pl.BlockSpec((1, G, D), lambda c, j: (c, 0, 0)),
            pl.BlockSpec((1, G, 1), lambda c, j: (c, 0, 0)),
        ],
        out_shape=[
            jax.ShapeDtypeStruct((2, G, D), _f32),
            jax.ShapeDtypeStruct((2, G, 1), _f32),
        ],
    )(acc, g, dis_col, b_all, batch_all)


def _head_body(ss_ref, cnt_ref, fcw_ref, fcb_ref, fw_ref, fb_ref, out_ref):
    o = None
    for cidx in range(2):
        m = ss_ref[cidx] / jnp.maximum(cnt_ref[cidx], 1.0)
        z = jnp.dot(m, fcw_ref[cidx], preferred_element_type=_f32) \
            + fcb_ref[cidx]
        z = jnp.where(z >= 0, z, 0.01 * z)
        fw = fw_ref[pl.ds(cidx * D, D), :]
        contrib = jnp.dot(z, fw, preferred_element_type=_f32)
        o = contrib if o is None else o + contrib
    out_ref[...] = o + fb_ref[...]


def _head_call(ss, cnt, fcw_all, fcb_all, final_w, final_b):
    return pl.pallas_call(
        _head_body,
        out_shape=jax.ShapeDtypeStruct((G, 1), _f32),
    )(ss, cnt, fcw_all, fcb_all, final_w, final_b)


def kernel(pro1_x, pro1_edge_index, pro1_batch, pro2_x, pro2_edge_index,
           pro2_batch, W1, b1, fc1_w, fc1_b, W2, b2, fc2_w, fc2_b,
           final_w, final_b):
    pad = jnp.full((EP - E,), N, _i32)
    s1 = jnp.concatenate([pro1_edge_index[0], pad])
    s2 = jnp.concatenate([pro2_edge_index[0], pad]) + NP
    d1 = jnp.concatenate([pro1_edge_index[1], pad])
    d2 = jnp.concatenate([pro2_edge_index[1], pad])
    srcs = jnp.stack([s1, s2]).reshape(2, NT, NCHUNK, CHUNK)
    dsts = jnp.stack([d1, d2]).reshape(2, NT, NCHUNK, CHUNK)
    rowid = jnp.arange(DEG_ROWS, dtype=_i32)

    x_all = jnp.pad(jnp.stack([pro1_x, pro2_x]), ((0, 0), (0, NP - N), (0, 0)))
    w_all = jnp.stack([W1, W2])
    b_all = jnp.stack([b1, b2]).reshape(2, 1, D)
    batch_all = jnp.pad(jnp.stack([pro1_batch, pro2_batch]),
                        ((0, 0), (0, NP - N)),
                        constant_values=G).reshape(2, NP, 1)
    fcw_all = jnp.stack([fc1_w, fc2_w])
    fcb_all = jnp.stack([fc1_b, fc2_b]).reshape(2, 1, D)
    fb = final_b.reshape(1, 1)

    deg2d = _deg_call(dsts, rowid)
    dis_col = _dis_call(deg2d).reshape(2, NP, 1)
    g = _g_call(x_all, dis_col, w_all)
    g_all = g.reshape(2 * NP, D)
    acc = _scatter_call(g_all, srcs, dsts)
    ss, cnt = _pool_call(acc, g, dis_col, b_all, batch_all)
    return _head_call(ss, cnt, fcw_all, fcb_all, final_w, fb)


# speedup vs baseline: 36.4822x; 2.2975x over previous
import jax
import jax.numpy as jnp
from jax import lax
from jax.experimental import pallas as pl
from jax.experimental.pallas import tpu as pltpu
from jax.experimental.pallas import tpu_sc as plsc

N = 10000
D = 128
E = 320000
G = 128
NP = 10240
NT = 16
CHUNK = 128
NCHUNK = 160
SLAB = 40
EP = NT * NCHUNK * CHUNK
ROWS_PER_TILE = NP // NT
DEG_ROWS = NP // 128

_f32 = jnp.float32
_i32 = jnp.int32


def _mesh():
    return plsc.VectorSubcoreMesh(core_axis_name="c", subcore_axis_name="s")


def _zero_rows(ref, nrows):
    zero16 = jnp.zeros((16,), _f32)

    def body(i, carry):
        for k in range(8):
            ref[i, pl.ds(k * 16, 16)] = zero16
        return carry

    lax.fori_loop(0, nrows, body, 0)


def _deg_body(dsts_hbm, rowid_hbm, deg_out, dst_v, deg1_v, deg2_v, rowid_v,
              deg_sh):
    c = lax.axis_index("c")
    s = lax.axis_index("s")
    zero16 = jnp.zeros((16,), _f32)

    def zbody(i, carry):
        deg1_v[pl.ds(i * 16, 16)] = zero16
        return carry

    lax.fori_loop(0, NP // 16, zbody, 0)
    _zero_rows(deg2_v, DEG_ROWS)

    @pl.when(s < 10)
    def _():
        pltpu.sync_copy(deg2_v.at[pl.ds(s * 8, 8)], deg_sh.at[pl.ds(s * 8, 8)])

    plsc.subcore_barrier()

    pltpu.sync_copy(dsts_hbm.at[c, s], dst_v)
    pltpu.sync_copy(rowid_hbm, rowid_v)
    ones16 = jnp.ones((16,), _f32)

    def body(j, carry):
        for k in range(8):
            idx = dst_v[j, pl.ds(k * 16, 16)]
            plsc.addupdate_scatter(deg1_v, [idx], ones16)
        return carry

    lax.fori_loop(0, NCHUNK, body, 0)

    def rbody(r, carry):
        for k in range(8):
            deg2_v[r, pl.ds(k * 16, 16)] = deg1_v[pl.ds(r * 128 + k * 16, 16)]
        return carry

    lax.fori_loop(0, DEG_ROWS, rbody, 0)
    pltpu.sync_copy(deg2_v, deg_sh.at[rowid_v], add=True)
    plsc.subcore_barrier()

    @pl.when(s < 10)
    def _():
        pltpu.sync_copy(deg_sh.at[pl.ds(s * 8, 8)],
                        deg_out.at[c, pl.ds(s * 8, 8)])


def _deg_call(dsts, rowid):
    return pl.kernel(
        _deg_body,
        out_type=jax.ShapeDtypeStruct((2, DEG_ROWS, 128), _f32),
        mesh=_mesh(),
        scratch_types=[
            pltpu.VMEM((NCHUNK, CHUNK), _i32),
            pltpu.VMEM((NP,), _f32),
            pltpu.VMEM((DEG_ROWS, 128), _f32),
            pltpu.VMEM((DEG_ROWS,), _i32),
            pltpu.VMEM_SHARED((DEG_ROWS, 128), _f32),
        ],
        compiler_params=pltpu.CompilerParams(needs_layout_passes=False),
    )(dsts, rowid)


def _scatter_body(g_hbm, srcs_hbm, dsts_hbm, acc_out, src_v, dst_v, row_a,
                  row_b, acc_sh, sem_a, sem_b):
    c = lax.axis_index("c")
    s = lax.axis_index("s")
    _zero_rows(row_a, CHUNK)
    for r in range(ROWS_PER_TILE // CHUNK):
        pltpu.sync_copy(row_a,
                        acc_sh.at[pl.ds(s * ROWS_PER_TILE + r * CHUNK, CHUNK)])
    plsc.subcore_barrier()

    def pair(i, carry):
        j = 2 * i
        pltpu.async_copy(g_hbm.at[src_v.at[j + 1]], row_b, sem_b)
        pltpu.make_async_copy(g_hbm.at[src_v.at[j]], row_a, sem_a).wait()
        pltpu.sync_copy(row_a, acc_sh.at[dst_v.at[j]], add=True)

        @pl.when(i < SLAB // 2 - 1)
        def _():
            pltpu.async_copy(g_hbm.at[src_v.at[j + 2]], row_a, sem_a)

        pltpu.make_async_copy(g_hbm.at[src_v.at[j + 1]], row_b, sem_b).wait()
        pltpu.sync_copy(row_b, acc_sh.at[dst_v.at[j + 1]], add=True)
        return carry

    for p in range(NCHUNK // SLAB):
        pltpu.sync_copy(srcs_hbm.at[c, s, pl.ds(p * SLAB, SLAB)], src_v)
        pltpu.sync_copy(dsts_hbm.at[c, s, pl.ds(p * SLAB, SLAB)], dst_v)
        pltpu.async_copy(g_hbm.at[src_v.at[0]], row_a, sem_a)
        lax.fori_loop(0, SLAB // 2, pair, 0)
    plsc.subcore_barrier()
    pltpu.sync_copy(acc_sh.at[pl.ds(s * ROWS_PER_TILE, ROWS_PER_TILE)],
                    acc_out.at[c, pl.ds(s * ROWS_PER_TILE, ROWS_PER_TILE)])


def _scatter_call(g_all, srcs, dsts):
    return pl.kernel(
        _scatter_body,
        out_type=jax.ShapeDtypeStruct((2, NP, D), _f32),
        mesh=_mesh(),
        scratch_types=[
            pltpu.VMEM((SLAB, CHUNK), _i32),
            pltpu.VMEM((SLAB, CHUNK), _i32),
            pltpu.VMEM((CHUNK, D), _f32),
            pltpu.VMEM((CHUNK, D), _f32),
            pltpu.VMEM_SHARED((NP, D), _f32),
            pltpu.SemaphoreType.DMA,
            pltpu.SemaphoreType.DMA,
        ],
        compiler_params=pltpu.CompilerParams(needs_layout_passes=False),
    )(g_all, srcs, dsts)


def _dis_body(deg_ref, dis_ref):
    deg = deg_ref[0]
    node = (lax.broadcasted_iota(_i32, (DEG_ROWS, 128), 0) * 128
            + lax.broadcasted_iota(_i32, (DEG_ROWS, 128), 1))
    dis_ref[0] = jnp.where(node < N, lax.rsqrt(deg + 1.0), 0.0)


def _dis_call(deg2d):
    return pl.pallas_call(
        _dis_body,
        grid=(2,),
        in_specs=[pl.BlockSpec((1, DEG_ROWS, 128), lambda c: (c, 0, 0))],
        out_specs=pl.BlockSpec((1, DEG_ROWS, 128), lambda c: (c, 0, 0)),
        out_shape=jax.ShapeDtypeStruct((2, DEG_ROWS, 128), _f32),
    )(deg2d)


_BLK = 512


def _g_body(x_ref, dis_ref, w_ref, g_ref):
    g_ref[0] = jnp.dot(x_ref[0] * dis_ref[0], w_ref[0],
                       preferred_element_type=_f32)


def _g_call(x_all, dis_col, w_all):
    return pl.pallas_call(
        _g_body,
        grid=(2, NP // _BLK),
        in_specs=[
            pl.BlockSpec((1, _BLK, D), lambda c, j: (c, j, 0)),
            pl.BlockSpec((1, _BLK, 1), lambda c, j: (c, j, 0)),
            pl.BlockSpec((1, D, D), lambda c, j: (c, 0, 0)),
        ],
        out_specs=pl.BlockSpec((1, _BLK, D), lambda c, j: (c, j, 0)),
        out_shape=jax.ShapeDtypeStruct((2, NP, D), _f32),
    )(x_all, dis_col, w_all)


def _pool_body(acc_ref, g_ref, dis_ref, b_ref, batch_ref, ss_ref, cnt_ref):
    j = pl.program_id(1)
    y = dis_ref[0] * (acc_ref[0] + g_ref[0]) + b_ref[0]
    y = jnp.where(y >= 0, y, 0.01 * y)
    gid = lax.broadcasted_iota(_i32, (1, G), 1)
    oh = (batch_ref[0] == gid).astype(_f32)
    ss = lax.dot_general(oh, y, (((0,), (0,)), ((), ())),
                         preferred_element_type=_f32)
    cn = lax.dot_general(oh, jnp.ones((_BLK, 1), _f32),
                         (((0,), (0,)), ((), ())),
                         preferred_element_type=_f32)

    @pl.when(j == 0)
    def _():
        ss_ref[0] = jnp.zeros((G, D), _f32)
        cnt_ref[0] = jnp.zeros((G, 1), _f32)

    ss_ref[0] += ss
    cnt_ref[0] += cn


def _pool_call(acc, g, dis_col, b_all, batch_all):
    return pl.pallas_call(
        _pool_body,
        grid=(2, NP // _BLK),
        in_specs=[
            pl.BlockSpec((1, _BLK, D), lambda c, j: (c, j, 0)),
            pl.BlockSpec((1, _BLK, D), lambda c, j: (c, j, 0)),
            pl.BlockSpec((1, _BLK, 1), lambda c, j: (c, j, 0)),
            pl.BlockSpec((1, 1, D), lambda c, j: (c, 0, 0)),
            pl.BlockSpec((1, _BLK, 1), lambda c, j: (c, j, 0)),
        ],
        out_specs=[
            pl.BlockSpec((1, G, D), lambda c, j: (c, 0, 0)),
            pl.BlockSpec((1, G, 1), lambda c, j: (c, 0, 0)),
        ],
        out_shape=[
            jax.ShapeDtypeStruct((2, G, D), _f32),
            jax.ShapeDtypeStruct((2, G, 1), _f32),
        ],
    )(acc, g, dis_col, b_all, batch_all)


def _head_body(ss_ref, cnt_ref, fcw_ref, fcb_ref, fw_ref, fb_ref, out_ref):
    o = None
    for cidx in range(2):
        m = ss_ref[cidx] / jnp.maximum(cnt_ref[cidx], 1.0)
        z = jnp.dot(m, fcw_ref[cidx], preferred_element_type=_f32) \
            + fcb_ref[cidx]
        z = jnp.where(z >= 0, z, 0.01 * z)
        fw = fw_ref[pl.ds(cidx * D, D), :]
        contrib = jnp.dot(z, fw, preferred_element_type=_f32)
        o = contrib if o is None else o + contrib
    out_ref[...] = o + fb_ref[...]


def _head_call(ss, cnt, fcw_all, fcb_all, final_w, final_b):
    return pl.pallas_call(
        _head_body,
        out_shape=jax.ShapeDtypeStruct((G, 1), _f32),
    )(ss, cnt, fcw_all, fcb_all, final_w, final_b)


def kernel(pro1_x, pro1_edge_index, pro1_batch, pro2_x, pro2_edge_index,
           pro2_batch, W1, b1, fc1_w, fc1_b, W2, b2, fc2_w, fc2_b,
           final_w, final_b):
    ar = jnp.arange(EP - E, dtype=_i32)
    pad_src = ar % NP
    pad_dst = N + ar % (NP - N)
    s1 = jnp.concatenate([pro1_edge_index[0], pad_src])
    s2 = jnp.concatenate([pro2_edge_index[0], pad_src]) + NP
    d1 = jnp.concatenate([pro1_edge_index[1], pad_dst])
    d2 = jnp.concatenate([pro2_edge_index[1], pad_dst])
    srcs = jnp.stack([s1, s2]).reshape(2, NT, NCHUNK, CHUNK)
    dsts = jnp.stack([d1, d2]).reshape(2, NT, NCHUNK, CHUNK)
    rowid = jnp.arange(DEG_ROWS, dtype=_i32)

    x_all = jnp.pad(jnp.stack([pro1_x, pro2_x]), ((0, 0), (0, NP - N), (0, 0)))
    w_all = jnp.stack([W1, W2])
    b_all = jnp.stack([b1, b2]).reshape(2, 1, D)
    batch_all = jnp.pad(jnp.stack([pro1_batch, pro2_batch]),
                        ((0, 0), (0, NP - N)),
                        constant_values=G).reshape(2, NP, 1)
    fcw_all = jnp.stack([fc1_w, fc2_w])
    fcb_all = jnp.stack([fc1_b, fc2_b]).reshape(2, 1, D)
    fb = final_b.reshape(1, 1)

    deg2d = _deg_call(dsts, rowid)
    dis_col = _dis_call(deg2d).reshape(2, NP, 1)
    g = _g_call(x_all, dis_col, w_all)
    g_all = g.reshape(2 * NP, D)
    acc = _scatter_call(g_all, srcs, dsts)
    ss, cnt = _pool_call(acc, g, dis_col, b_all, batch_all)
    return _head_call(ss, cnt, fcw_all, fcb_all, final_w, fb)
